# Initial kernel scaffold; baseline (speedup 1.0000x reference)
import jax
import jax.numpy as jnp
from jax.experimental import pallas as pl


def kernel(target_ids, pos_context_ids, neg_context_ids, W_in, W_out):
    raise NotImplementedError("write your pallas kernel here")


# speedup vs baseline: 2.5618x; 2.5618x over previous
import functools

import jax
import jax.numpy as jnp
from jax import lax
from jax.experimental import pallas as pl
from jax.experimental.pallas import tpu as pltpu
from jax.experimental.pallas import tpu_sc as plsc

VOCAB = 100000
EMBED = 128
BATCH = 16384
NEG_K = 200
KP = 208
KH = KP // 2
NCORES = 2
NSUB = 16
NW = NCORES * NSUB
BPW = BATCH // NW
BCH = 64
NCH = BPW // BCH
NLANE = 16
ECH = EMBED // NLANE


def _sc_body(tgt_hbm, ids_hbm, win_hbm, wout_hbm, out_hbm,
             ids_v, tgt_v, vc_v, rows_v, score_v, trans_v,
             sem_rows, sem_vc):
    c = lax.axis_index("c")
    s = lax.axis_index("s")
    w = s * NCORES + c
    b0 = w * BPW
    lanes = lax.iota(jnp.int32, NLANE)

    @pl.loop(0, NCH)
    def _chunk(ci):
        bb = b0 + ci * BCH
        pltpu.sync_copy(ids_hbm.at[pl.ds(bb, BCH)], ids_v)
        pltpu.sync_copy(tgt_hbm.at[pl.ds(bb, BCH)], tgt_v)
        pltpu.async_copy(win_hbm.at[tgt_v], vc_v, sem_vc).wait()

        @pl.loop(0, BCH)
        def _b(bi):
            cp0 = pltpu.async_copy(wout_hbm.at[ids_v.at[bi, 0]],
                                   rows_v.at[pl.ds(0, KH)], sem_rows)
            cp1 = pltpu.async_copy(wout_hbm.at[ids_v.at[bi, 1]],
                                   rows_v.at[pl.ds(KH, KH)], sem_rows)
            cp0.wait()
            cp1.wait()
            vc = [vc_v[bi, pl.ds(e * NLANE, NLANE)] for e in range(ECH)]

            @pl.loop(0, KP // NLANE)
            def _kg(kg):
                for j in range(NLANE):
                    k = kg * NLANE + j
                    p = rows_v[k, pl.ds(0, NLANE)] * vc[0]
                    for e in range(1, ECH):
                        p = p + rows_v[k, pl.ds(e * NLANE, NLANE)] * vc[e]
                    plsc.store_scatter(
                        trans_v, [lanes * NLANE + j], p)
                acc = trans_v[pl.ds(0, NLANE)]
                for l in range(1, NLANE):
                    acc = acc + trans_v[pl.ds(l * NLANE, NLANE)]
                score_v[bi, pl.ds(kg * NLANE, NLANE)] = acc

        pltpu.sync_copy(score_v, out_hbm.at[pl.ds(bb, BCH)])


def _sc_scores(tgt, ids, W_in, W_out):
    mesh = plsc.VectorSubcoreMesh(core_axis_name="c", subcore_axis_name="s")
    kern = pl.kernel(
        _sc_body,
        out_type=jax.ShapeDtypeStruct((BATCH, KP), jnp.float32),
        mesh=mesh,
        scratch_types=[
            pltpu.VMEM((BCH, 2, KH), jnp.int32),
            pltpu.VMEM((BCH,), jnp.int32),
            pltpu.VMEM((BCH, EMBED), jnp.float32),
            pltpu.VMEM((KP, EMBED), jnp.float32),
            pltpu.VMEM((BCH, KP), jnp.float32),
            pltpu.VMEM((NLANE * NLANE,), jnp.float32),
            pltpu.SemaphoreType.DMA,
            pltpu.SemaphoreType.DMA,
        ],
        compiler_params=pltpu.CompilerParams(needs_layout_passes=False),
    )
    return kern(tgt, ids, W_in, W_out)


def _tc_loss(scores):
    BLK = 2048
    grid = BATCH // BLK

    def body(s_ref, o_ref):
        i = pl.program_id(0)
        sc = s_ref[...]
        col = lax.broadcasted_iota(jnp.int32, (BLK, KP), 1)
        t = jnp.log1p(jnp.exp(-jnp.abs(sc)))
        extra = jnp.where(col == 0, jnp.maximum(-sc, 0.0),
                          jnp.maximum(sc, 0.0))
        elem = jnp.where(col < 1 + NEG_K, t + extra, 0.0)
        part = jnp.sum(elem)

        @pl.when(i == 0)
        def _():
            o_ref[0, 0] = 0.0

        o_ref[0, 0] += part

    out = pl.pallas_call(
        body,
        grid=(grid,),
        in_specs=[pl.BlockSpec((BLK, KP), lambda i: (i, 0))],
        out_specs=pl.BlockSpec(memory_space=pltpu.SMEM),
        out_shape=jax.ShapeDtypeStruct((1, 1), jnp.float32),
    )(scores)
    return out[0, 0] / BATCH


def kernel(target_ids, pos_context_ids, neg_context_ids, W_in, W_out):
    tgt = target_ids.astype(jnp.int32)
    pad = jnp.zeros((BATCH, KP - 1 - NEG_K), jnp.int32)
    ids = jnp.concatenate(
        [pos_context_ids.astype(jnp.int32)[:, None],
         neg_context_ids.astype(jnp.int32), pad], axis=1)
    ids = ids.reshape(BATCH, 2, KH)
    scores = _sc_scores(tgt, ids, W_in, W_out)
    return _tc_loss(scores)
